# baseline (speedup 1.0000x reference)
import functools
import math

import jax
import jax.numpy as jnp
from jax import lax
from jax.experimental import pallas as pl
from jax.experimental.pallas import tpu as pltpu
from jax.experimental.pallas import tpu_sc as plsc

_NUM_POS_FEATS = 384
_TEMPERATURE = 10000.0
_TBLK = 512
_ROW = 4 * _NUM_POS_FEATS


def _table_block(t_ref):
    i = pl.program_id(0)
    blk, _ = t_ref.shape
    nf = _NUM_POS_FEATS
    k = i * blk + jax.lax.broadcasted_iota(jnp.int32, (blk, nf), 0)
    j = jax.lax.broadcasted_iota(jnp.int32, (blk, nf), 1)
    inv_dim_t = jnp.exp(
        (-math.log(_TEMPERATURE) * 2.0 / nf) * ((j // 2).astype(jnp.float32))
    )
    phase = k.astype(jnp.float32) * inv_dim_t
    z = jnp.zeros((blk, 2 * nf), jnp.float32)
    t_ref[...] = jnp.concatenate([jnp.sin(phase), z, jnp.cos(phase)], axis=1)


def _make_sc_gather(V, B, D):
    info = plsc.get_sparse_core_info()
    NC, NS = info.num_cores, info.num_subcores
    NW = NC * NS
    b_per_w = B // NW
    C = 64
    n_chunks = b_per_w // C
    mesh = plsc.VectorSubcoreMesh(core_axis_name="c", subcore_axis_name="s")

    @functools.partial(
        pl.kernel,
        mesh=mesh,
        out_type=jax.ShapeDtypeStruct((B, D), jnp.float32),
        scratch_types=[
            pltpu.VMEM((C,), jnp.int32),
            pltpu.VMEM((C, D), jnp.float32),
            pltpu.SemaphoreType.DMA,
        ],
    )
    def sc_gather(table_hbm, idx_hbm, out_hbm, idx_v, rows_v, sem):
        wid = lax.axis_index("s") * NC + lax.axis_index("c")
        base = wid * b_per_w

        def body(c, carry):
            off = base + c * C
            pltpu.sync_copy(idx_hbm.at[pl.ds(off, C)], idx_v)
            pltpu.async_copy(table_hbm.at[idx_v], rows_v, sem).wait()
            pltpu.sync_copy(rows_v, out_hbm.at[pl.ds(off, C)])
            return carry

        lax.fori_loop(0, n_chunks, body, 0)

    return sc_gather


@functools.partial(jax.jit, static_argnames=())
def kernel(x):
    batch, seq = x.shape
    nf = _NUM_POS_FEATS
    V = seq // 2
    B = batch * V
    table = pl.pallas_call(
        _table_block,
        grid=(V // _TBLK,),
        out_shape=jax.ShapeDtypeStruct((V, _ROW), jnp.float32),
        out_specs=pl.BlockSpec((_TBLK, _ROW), lambda i: (i, 0)),
    )()
    idx = jnp.arange(B, dtype=jnp.int32) % V
    out_flat = _make_sc_gather(V, B, _ROW)(table, idx)
    return out_flat.reshape(batch, seq, 2 * nf)


# speedup vs baseline: 6.7514x; 6.7514x over previous
import functools
import math

import jax
import jax.numpy as jnp
from jax.experimental import pallas as pl
from jax.experimental.pallas import tpu as pltpu

_NUM_POS_FEATS = 384
_TEMPERATURE = 10000.0
_BLK = 256


def _pos_embed_block(o_ref, sinb_ref, cosb_ref):
    i = pl.program_id(0)
    batch, blk, _ = o_ref.shape
    nf = _NUM_POS_FEATS
    kblk = blk // 2

    j = jax.lax.broadcasted_iota(jnp.int32, (blk, nf), 1)
    inv_dim_t = jnp.exp(
        (-math.log(_TEMPERATURE) * 2.0 / nf) * ((j // 2).astype(jnp.float32))
    )

    @pl.when(i == 0)
    def _init():
        p = jax.lax.broadcasted_iota(jnp.int32, (blk, nf), 0)
        base = (p // 2).astype(jnp.float32) * inv_dim_t
        sinb_ref[...] = jnp.sin(base)
        cosb_ref[...] = jnp.cos(base)

    phase_a = (i * kblk) * inv_dim_t[:8, :]
    sin_a = jnp.sin(phase_a)[:1]
    cos_a = jnp.cos(phase_a)[:1]

    sinb = sinb_ref[...]
    cosb = cosb_ref[...]
    sin_k = sinb * cos_a + cosb * sin_a
    cos_k = cosb * cos_a - sinb * sin_a

    p = jax.lax.broadcasted_iota(jnp.int32, (blk, nf), 0)
    even = (p % 2) == 0
    sin_half = jnp.where(even, sin_k, 0.0)
    cos_half = jnp.where(even, 0.0, cos_k)
    full = jnp.concatenate([sin_half, cos_half], axis=1)
    o_ref[...] = jnp.broadcast_to(full[None], (batch, blk, 2 * nf))


@functools.partial(jax.jit, static_argnames=())
def kernel(x):
    batch, seq = x.shape
    nf2 = 2 * _NUM_POS_FEATS
    grid = (seq // _BLK,)
    return pl.pallas_call(
        _pos_embed_block,
        grid=grid,
        out_shape=jax.ShapeDtypeStruct((batch, seq, nf2), jnp.float32),
        out_specs=pl.BlockSpec((batch, _BLK, nf2), lambda i: (0, i, 0)),
        scratch_shapes=[
            pltpu.VMEM((_BLK, _NUM_POS_FEATS), jnp.float32),
            pltpu.VMEM((_BLK, _NUM_POS_FEATS), jnp.float32),
        ],
    )()
